# baseline (speedup 1.0000x reference)
import functools
import jax
import jax.numpy as jnp
from jax import lax
from jax.experimental import pallas as pl
from jax.experimental.pallas import tpu as pltpu
from jax.experimental.pallas import tpu_sc as plsc

_NC, _NS, _L = 2, 16, 16
_NW = _NC * _NS
_N = 32 * 1024
_U = 8


def kernel(inputs, pos_table):
    B, S, D = inputs.shape
    W = S * D
    chunk = W // _NW
    n_groups = chunk // _N

    x2 = inputs.reshape(B, W)
    p1 = pos_table.reshape(W)

    mesh = plsc.VectorSubcoreMesh(core_axis_name="c", subcore_axis_name="s")

    @functools.partial(
        pl.kernel,
        out_type=jax.ShapeDtypeStruct((B, W), jnp.float32),
        mesh=mesh,
        scratch_types=[
            pltpu.VMEM((_N,), jnp.float32),
            pltpu.VMEM((_N,), jnp.float32),
        ],
    )
    def sc_add(in_hbm, pos_hbm, out_hbm, in_v, pos_v):
        wid = lax.axis_index("s") * _NC + lax.axis_index("c")
        base = wid * chunk

        def g_body(g, _):
            off = base + g * _N
            pltpu.sync_copy(pos_hbm.at[pl.ds(off, _N)], pos_v)

            def b_body(b, _):
                pltpu.sync_copy(in_hbm.at[b, pl.ds(off, _N)], in_v)

                def add_body(i, _):
                    for u in range(_U):
                        sl = pl.ds((i * _U + u) * _L, _L)
                        in_v[sl] = in_v[sl] + pos_v[sl]
                    return _

                lax.fori_loop(0, _N // (_L * _U), add_body, 0)
                pltpu.sync_copy(in_v, out_hbm.at[b, pl.ds(off, _N)])
                return _

            lax.fori_loop(0, B, b_body, 0)
            return _

        lax.fori_loop(0, n_groups, g_body, 0)

    out = sc_add(x2, p1)
    return out.reshape(B, S, D)


# speedup vs baseline: 4.7306x; 4.7306x over previous
import jax
import jax.numpy as jnp
from jax.experimental import pallas as pl
from jax.experimental.pallas import tpu as pltpu

_SEQ_BLOCK = 2048


def _add_kernel(x_ref, p_ref, o_ref):
    o_ref[...] = x_ref[...] + p_ref[...]


def kernel(inputs, pos_table):
    B, S, D = inputs.shape
    n_seq = S // _SEQ_BLOCK
    return pl.pallas_call(
        _add_kernel,
        grid=(n_seq, B),
        in_specs=[
            pl.BlockSpec((1, _SEQ_BLOCK, D), lambda s, b: (b, s, 0)),
            pl.BlockSpec((_SEQ_BLOCK, D), lambda s, b: (s, 0)),
        ],
        out_specs=pl.BlockSpec((1, _SEQ_BLOCK, D), lambda s, b: (b, s, 0)),
        out_shape=jax.ShapeDtypeStruct((B, S, D), inputs.dtype),
        compiler_params=pltpu.CompilerParams(
            dimension_semantics=("parallel", "arbitrary"),
            vmem_limit_bytes=128 * 1024 * 1024,
        ),
    )(inputs, pos_table)
